# baseline (speedup 1.0000x reference)
import functools

import jax
import jax.numpy as jnp
from jax import lax
from jax.experimental import pallas as pl
from jax.experimental.pallas import tpu as pltpu
from jax.experimental.pallas import tpu_sc as plsc

_NW = 32
_LANES = 16
_CW = 72


def _sc_counts(pos_t, dep_t, B, L):
    R = B // _NW
    CH = R // 2
    GC = CH // _LANES
    mesh = plsc.VectorSubcoreMesh(core_axis_name="c", subcore_axis_name="s")

    @functools.partial(
        pl.kernel,
        out_type=jax.ShapeDtypeStruct((_CW, B), jnp.float32),
        mesh=mesh,
        compiler_params=pltpu.CompilerParams(needs_layout_passes=False),
        scratch_types=[
            pltpu.VMEM((2, L, CH), jnp.int32),
            pltpu.VMEM((2, L, CH), jnp.int32),
            pltpu.VMEM((_CW, R), jnp.float32),
            pltpu.SemaphoreType.DMA,
            pltpu.SemaphoreType.DMA,
            pltpu.SemaphoreType.DMA,
        ],
    )
    def k(pos_hbm, dep_hbm, out_hbm, pos_v, dep_v, cnt_v, sem0, sem1, osem):
        wid = lax.axis_index("s") * 2 + lax.axis_index("c")
        base = wid * R
        sems = (sem0, sem1)
        pending = []
        for b in range(2):
            cb = base + b * CH
            pending.append((
                pltpu.async_copy(pos_hbm.at[:, pl.ds(cb, CH)], pos_v.at[b], sems[b]),
                pltpu.async_copy(dep_hbm.at[:, pl.ds(cb, CH)], dep_v.at[b], sems[b]),
            ))

        ones = jnp.full((_LANES,), 1.0, jnp.float32)
        zeros = jnp.zeros((_LANES,), jnp.float32)
        iota = lax.iota(jnp.int32, _LANES)

        @plsc.parallel_loop(0, _CW, unroll=1)
        def zrow(c):
            for j in range(R // _LANES):
                cnt_v[c, pl.ds(j * _LANES, _LANES)] = zeros

        outs = []
        for b in range(2):
            for h in pending[b]:
                h.wait()

            @plsc.parallel_loop(0, GC, unroll=1)
            def grp(g):
                gb = g * _LANES
                rows = b * CH + gb + iota

                def lchunk(lc, carry):
                    lb = lc * (L // 2)
                    for k in range(L // 2):
                        idx = pos_v[b, lb + k, pl.ds(gb, _LANES)]
                        plsc.addupdate_scatter(cnt_v, [idx, rows], ones)
                        idxd = dep_v[b, lb + k, pl.ds(gb, _LANES)] + 19
                        plsc.addupdate_scatter(cnt_v, [idxd, rows], ones)
                    return carry

                lax.fori_loop(0, 2, lchunk, 0)

            outs.append(pltpu.async_copy(
                cnt_v.at[:, pl.ds(b * CH, CH)],
                out_hbm.at[:, pl.ds(base + b * CH, CH)],
                osem,
            ))
        for h in outs:
            h.wait()

    return k(pos_t, dep_t)


def _tc_matmul(counts_t, w_pos, w_dep, B):
    BLK = 2048

    def body(c_ref, wp_ref, wd_ref, po_ref, do_ref):
        c = c_ref[...]
        dn = (((0,), (0,)), ((), ()))
        po_ref[...] = lax.dot_general(
            wp_ref[...], c, dn, preferred_element_type=jnp.float32
        )
        do_ref[...] = lax.dot_general(
            wd_ref[...], c, dn, preferred_element_type=jnp.float32
        )

    return pl.pallas_call(
        body,
        grid=(B // BLK,),
        in_specs=[
            pl.BlockSpec((_CW, BLK), lambda i: (0, i)),
            pl.BlockSpec((_CW, 64), lambda i: (0, 0)),
            pl.BlockSpec((_CW, 64), lambda i: (0, 0)),
        ],
        out_specs=[
            pl.BlockSpec((64, BLK), lambda i: (0, i)),
            pl.BlockSpec((64, BLK), lambda i: (0, i)),
        ],
        out_shape=[
            jax.ShapeDtypeStruct((64, B), jnp.float32),
            jax.ShapeDtypeStruct((64, B), jnp.float32),
        ],
    )(counts_t, w_pos, w_dep)


def kernel(padded_pos, padded_deprel, pos_table, deprel_table):
    B, L = padded_pos.shape
    counts_t = _sc_counts(padded_pos.T, padded_deprel.T, B, L)
    w_pos = jnp.zeros((_CW, 64), jnp.float32).at[: pos_table.shape[0]].set(pos_table)
    w_dep = (
        jnp.zeros((_CW, 64), jnp.float32)
        .at[19 : 19 + deprel_table.shape[0]]
        .set(deprel_table)
    )
    po_t, do_t = _tc_matmul(counts_t, w_pos, w_dep, B)
    return (po_t.T, do_t.T)


# speedup vs baseline: 1.1681x; 1.1681x over previous
import functools

import jax
import jax.numpy as jnp
from jax import lax
from jax.experimental import pallas as pl
from jax.experimental.pallas import tpu as pltpu
from jax.experimental.pallas import tpu_sc as plsc

_NW = 32
_LANES = 16
_CW = 72


def _sc_counts(pos_t, dep_t, B, L):
    R = B // _NW
    CH = R // 2
    GC = CH // _LANES
    mesh = plsc.VectorSubcoreMesh(core_axis_name="c", subcore_axis_name="s")

    @functools.partial(
        pl.kernel,
        out_type=jax.ShapeDtypeStruct((_CW, B), jnp.float32),
        mesh=mesh,
        compiler_params=pltpu.CompilerParams(needs_layout_passes=False),
        scratch_types=[
            pltpu.VMEM((2, L, CH), jnp.int32),
            pltpu.VMEM((2, L, CH), jnp.int32),
            pltpu.VMEM((24, R), jnp.float32),
            pltpu.VMEM((48, R), jnp.float32),
            pltpu.SemaphoreType.DMA,
            pltpu.SemaphoreType.DMA,
            pltpu.SemaphoreType.DMA,
        ],
    )
    def k(pos_hbm, dep_hbm, out_hbm, pos_v, dep_v, cnp_v, cnd_v, sem0, sem1, osem):
        wid = lax.axis_index("s") * 2 + lax.axis_index("c")
        base = wid * R
        sems = (sem0, sem1)
        pending = []
        for b in range(2):
            cb = base + b * CH
            pending.append((
                pltpu.async_copy(pos_hbm.at[:, pl.ds(cb, CH)], pos_v.at[b], sems[b]),
                pltpu.async_copy(dep_hbm.at[:, pl.ds(cb, CH)], dep_v.at[b], sems[b]),
            ))

        ones = jnp.full((_LANES,), 1.0, jnp.float32)
        zeros = jnp.zeros((_LANES,), jnp.float32)
        iota = lax.iota(jnp.int32, _LANES)

        @plsc.parallel_loop(0, 24, unroll=1)
        def zrowp(c):
            for j in range(R // _LANES):
                cnp_v[c, pl.ds(j * _LANES, _LANES)] = zeros

        @plsc.parallel_loop(0, 48, unroll=1)
        def zrowd(c):
            for j in range(R // _LANES):
                cnd_v[c, pl.ds(j * _LANES, _LANES)] = zeros

        outs = []
        for b in range(2):
            for h in pending[b]:
                h.wait()

            @plsc.parallel_loop(0, GC, unroll=1)
            def grp(g):
                gb = g * _LANES
                rows = b * CH + gb + iota
                for l in range(L):
                    idx = pos_v[b, l, pl.ds(gb, _LANES)]
                    plsc.addupdate_scatter(cnp_v, [idx, rows], ones)
                    idxd = dep_v[b, l, pl.ds(gb, _LANES)]
                    plsc.addupdate_scatter(cnd_v, [idxd, rows], ones)

            outs.append(pltpu.async_copy(
                cnp_v.at[:, pl.ds(b * CH, CH)],
                out_hbm.at[pl.ds(0, 24), pl.ds(base + b * CH, CH)],
                osem,
            ))
            outs.append(pltpu.async_copy(
                cnd_v.at[:, pl.ds(b * CH, CH)],
                out_hbm.at[pl.ds(24, 48), pl.ds(base + b * CH, CH)],
                osem,
            ))
        for h in outs:
            h.wait()

    return k(pos_t, dep_t)


def _tc_matmul(counts_t, w_pos, w_dep, B):
    BLK = 2048

    def body(c_ref, wp_ref, wd_ref, po_ref, do_ref):
        c = c_ref[...]
        dn = (((0,), (0,)), ((), ()))
        po_ref[...] = lax.dot_general(
            wp_ref[...], c, dn, preferred_element_type=jnp.float32
        )
        do_ref[...] = lax.dot_general(
            wd_ref[...], c, dn, preferred_element_type=jnp.float32
        )

    return pl.pallas_call(
        body,
        grid=(B // BLK,),
        in_specs=[
            pl.BlockSpec((_CW, BLK), lambda i: (0, i)),
            pl.BlockSpec((_CW, 64), lambda i: (0, 0)),
            pl.BlockSpec((_CW, 64), lambda i: (0, 0)),
        ],
        out_specs=[
            pl.BlockSpec((64, BLK), lambda i: (0, i)),
            pl.BlockSpec((64, BLK), lambda i: (0, i)),
        ],
        out_shape=[
            jax.ShapeDtypeStruct((64, B), jnp.float32),
            jax.ShapeDtypeStruct((64, B), jnp.float32),
        ],
    )(counts_t, w_pos, w_dep)


def kernel(padded_pos, padded_deprel, pos_table, deprel_table):
    B, L = padded_pos.shape
    counts_t = _sc_counts(padded_pos.T, padded_deprel.T, B, L)
    w_pos = jnp.zeros((_CW, 64), jnp.float32).at[: pos_table.shape[0]].set(pos_table)
    w_dep = (
        jnp.zeros((_CW, 64), jnp.float32)
        .at[24 : 24 + deprel_table.shape[0]]
        .set(deprel_table)
    )
    po_t, do_t = _tc_matmul(counts_t, w_pos, w_dep, B)
    return (po_t.T, do_t.T)


# speedup vs baseline: 1.2165x; 1.0414x over previous
import functools

import jax
import jax.numpy as jnp
from jax import lax
from jax.experimental import pallas as pl
from jax.experimental.pallas import tpu as pltpu
from jax.experimental.pallas import tpu_sc as plsc

_NW = 32
_LANES = 16
_CW = 72


def _sc_counts(pos_t, dep_t, B, L):
    R = B // _NW
    CH = R // 2
    GC = CH // _LANES
    mesh = plsc.VectorSubcoreMesh(core_axis_name="c", subcore_axis_name="s")

    @functools.partial(
        pl.kernel,
        out_type=jax.ShapeDtypeStruct((_CW, B), jnp.float32),
        mesh=mesh,
        compiler_params=pltpu.CompilerParams(needs_layout_passes=False),
        scratch_types=[
            pltpu.VMEM((2, L, CH), jnp.int32),
            pltpu.VMEM((2, L, CH), jnp.int32),
            pltpu.VMEM((24, R), jnp.float32),
            pltpu.VMEM((48, R), jnp.float32),
            pltpu.SemaphoreType.DMA,
            pltpu.SemaphoreType.DMA,
            pltpu.SemaphoreType.DMA,
        ],
    )
    def k(pos_hbm, dep_hbm, out_hbm, pos_v, dep_v, cnp_v, cnd_v, sem0, sem1, osem):
        wid = lax.axis_index("s") * 2 + lax.axis_index("c")
        base = wid * R
        sems = (sem0, sem1)
        pending = []
        for b in range(2):
            cb = base + b * CH
            pending.append((
                pltpu.async_copy(pos_hbm.at[:, pl.ds(cb, CH)], pos_v.at[b], sems[b]),
                pltpu.async_copy(dep_hbm.at[:, pl.ds(cb, CH)], dep_v.at[b], sems[b]),
            ))

        ones = jnp.full((_LANES,), 1.0, jnp.float32)
        zeros = jnp.zeros((_LANES,), jnp.float32)
        iota = lax.iota(jnp.int32, _LANES)

        @plsc.parallel_loop(0, 24, unroll=1)
        def zrowp(c):
            for j in range(R // _LANES):
                cnp_v[c, pl.ds(j * _LANES, _LANES)] = zeros

        @plsc.parallel_loop(0, 48, unroll=1)
        def zrowd(c):
            for j in range(R // _LANES):
                cnd_v[c, pl.ds(j * _LANES, _LANES)] = zeros

        outs = []
        for b in range(2):
            for h in pending[b]:
                h.wait()

            @plsc.parallel_loop(0, GC, unroll=1)
            def grp(g):
                gb = g * _LANES
                rows = b * CH + gb + iota
                for l in range(L):
                    idx = pos_v[b, l, pl.ds(gb, _LANES)]
                    plsc.addupdate_scatter(cnp_v, [idx, rows], ones)
                    idxd = dep_v[b, l, pl.ds(gb, _LANES)]
                    plsc.addupdate_scatter(cnd_v, [idxd, rows], ones)

            outs.append(pltpu.async_copy(
                cnp_v.at[:, pl.ds(b * CH, CH)],
                out_hbm.at[pl.ds(0, 24), pl.ds(base + b * CH, CH)],
                osem,
            ))
            outs.append(pltpu.async_copy(
                cnd_v.at[:, pl.ds(b * CH, CH)],
                out_hbm.at[pl.ds(24, 48), pl.ds(base + b * CH, CH)],
                osem,
            ))
        for h in outs:
            h.wait()

    return k(pos_t, dep_t)


def _tc_matmul(counts_t, w_pos, w_dep, B):
    BLK = 4096

    def body(c_ref, wp_ref, wd_ref, po_ref, do_ref):
        c = c_ref[...]
        dn = (((0,), (0,)), ((), ()))
        po_ref[...] = lax.dot_general(
            wp_ref[...], c, dn, preferred_element_type=jnp.float32
        )
        do_ref[...] = lax.dot_general(
            wd_ref[...], c, dn, preferred_element_type=jnp.float32
        )

    return pl.pallas_call(
        body,
        grid=(B // BLK,),
        compiler_params=pltpu.CompilerParams(
            dimension_semantics=("arbitrary",)
        ),
        in_specs=[
            pl.BlockSpec((_CW, BLK), lambda i: (0, i)),
            pl.BlockSpec((_CW, 64), lambda i: (0, 0)),
            pl.BlockSpec((_CW, 64), lambda i: (0, 0)),
        ],
        out_specs=[
            pl.BlockSpec((64, BLK), lambda i: (0, i)),
            pl.BlockSpec((64, BLK), lambda i: (0, i)),
        ],
        out_shape=[
            jax.ShapeDtypeStruct((64, B), jnp.float32),
            jax.ShapeDtypeStruct((64, B), jnp.float32),
        ],
    )(counts_t, w_pos, w_dep)


def kernel(padded_pos, padded_deprel, pos_table, deprel_table):
    B, L = padded_pos.shape
    counts_t = _sc_counts(padded_pos.T, padded_deprel.T, B, L)
    w_pos = jnp.zeros((_CW, 64), jnp.float32).at[: pos_table.shape[0]].set(pos_table)
    w_dep = (
        jnp.zeros((_CW, 64), jnp.float32)
        .at[24 : 24 + deprel_table.shape[0]]
        .set(deprel_table)
    )
    po_t, do_t = _tc_matmul(counts_t, w_pos, w_dep, B)
    return (po_t.T, do_t.T)


# speedup vs baseline: 1.2584x; 1.0344x over previous
import functools

import jax
import jax.numpy as jnp
from jax import lax
from jax.experimental import pallas as pl
from jax.experimental.pallas import tpu as pltpu
from jax.experimental.pallas import tpu_sc as plsc

_NW = 32
_LANES = 16
_CW = 72


def _sc_counts(pos_t, dep_t, B, L):
    R = B // _NW
    CH = R // 2
    GC = CH // _LANES
    mesh = plsc.VectorSubcoreMesh(core_axis_name="c", subcore_axis_name="s")

    @functools.partial(
        pl.kernel,
        out_type=jax.ShapeDtypeStruct((_CW, B), jnp.float32),
        mesh=mesh,
        compiler_params=pltpu.CompilerParams(needs_layout_passes=False),
        scratch_types=[
            pltpu.VMEM((2, L, CH), jnp.int32),
            pltpu.VMEM((2, L, CH), jnp.int32),
            pltpu.VMEM((24, R), jnp.float32),
            pltpu.VMEM((48, R), jnp.float32),
            pltpu.SemaphoreType.DMA,
            pltpu.SemaphoreType.DMA,
            pltpu.SemaphoreType.DMA,
        ],
    )
    def k(pos_hbm, dep_hbm, out_hbm, pos_v, dep_v, cnp_v, cnd_v, sem0, sem1, osem):
        wid = lax.axis_index("s") * 2 + lax.axis_index("c")
        base = wid * R
        sems = (sem0, sem1)
        pending = []
        for b in range(2):
            cb = base + b * CH
            pending.append((
                pltpu.async_copy(pos_hbm.at[:, pl.ds(cb, CH)], pos_v.at[b], sems[b]),
                pltpu.async_copy(dep_hbm.at[:, pl.ds(cb, CH)], dep_v.at[b], sems[b]),
            ))

        ones = jnp.full((_LANES,), 1.0, jnp.float32)
        zeros = jnp.zeros((_LANES,), jnp.float32)
        iota = lax.iota(jnp.int32, _LANES)

        @plsc.parallel_loop(0, 24, unroll=1)
        def zrowp(c):
            for j in range(R // _LANES):
                cnp_v[c, pl.ds(j * _LANES, _LANES)] = zeros

        @plsc.parallel_loop(0, 48, unroll=1)
        def zrowd(c):
            for j in range(R // _LANES):
                cnd_v[c, pl.ds(j * _LANES, _LANES)] = zeros

        outs = []
        for b in range(2):
            for h in pending[b]:
                h.wait()

            @plsc.parallel_loop(0, GC, unroll=1)
            def grp(g):
                gb = g * _LANES
                rows = b * CH + gb + iota
                for l in range(L):
                    idx = pos_v[b, l, pl.ds(gb, _LANES)]
                    plsc.addupdate_scatter(cnp_v, [idx, rows], ones)
                    idxd = dep_v[b, l, pl.ds(gb, _LANES)]
                    plsc.addupdate_scatter(cnd_v, [idxd, rows], ones)

            outs.append(pltpu.async_copy(
                cnp_v.at[:, pl.ds(b * CH, CH)],
                out_hbm.at[pl.ds(0, 24), pl.ds(base + b * CH, CH)],
                osem,
            ))
            outs.append(pltpu.async_copy(
                cnd_v.at[:, pl.ds(b * CH, CH)],
                out_hbm.at[pl.ds(24, 48), pl.ds(base + b * CH, CH)],
                osem,
            ))
        for h in outs:
            h.wait()

    return k(pos_t, dep_t)


def _tc_matmul(counts_t, w_pos, w_dep, B):
    BLK = 8192

    def body(c_ref, wp_ref, wd_ref, po_ref, do_ref):
        c = c_ref[...]
        dn = (((0,), (0,)), ((), ()))
        po_ref[...] = lax.dot_general(
            wp_ref[...], c, dn, preferred_element_type=jnp.float32
        )
        do_ref[...] = lax.dot_general(
            wd_ref[...], c, dn, preferred_element_type=jnp.float32
        )

    return pl.pallas_call(
        body,
        grid=(B // BLK,),
        compiler_params=pltpu.CompilerParams(
            dimension_semantics=("arbitrary",)
        ),
        in_specs=[
            pl.BlockSpec((_CW, BLK), lambda i: (0, i)),
            pl.BlockSpec((_CW, 64), lambda i: (0, 0)),
            pl.BlockSpec((_CW, 64), lambda i: (0, 0)),
        ],
        out_specs=[
            pl.BlockSpec((64, BLK), lambda i: (0, i)),
            pl.BlockSpec((64, BLK), lambda i: (0, i)),
        ],
        out_shape=[
            jax.ShapeDtypeStruct((64, B), jnp.float32),
            jax.ShapeDtypeStruct((64, B), jnp.float32),
        ],
    )(counts_t, w_pos, w_dep)


def kernel(padded_pos, padded_deprel, pos_table, deprel_table):
    B, L = padded_pos.shape
    counts_t = _sc_counts(padded_pos.T, padded_deprel.T, B, L)
    w_pos = jnp.zeros((_CW, 64), jnp.float32).at[: pos_table.shape[0]].set(pos_table)
    w_dep = (
        jnp.zeros((_CW, 64), jnp.float32)
        .at[24 : 24 + deprel_table.shape[0]]
        .set(deprel_table)
    )
    po_t, do_t = _tc_matmul(counts_t, w_pos, w_dep, B)
    return (po_t.T, do_t.T)
